# baseline (speedup 1.0000x reference)
import functools

import jax
import jax.numpy as jnp
from jax import lax
from jax.experimental import pallas as pl
from jax.experimental.pallas import tpu as pltpu

ACT = 1024
DICT = 16384
TOKENS = 4096
TOPK = 64
L1_COEFF = 0.0008

TB = 256
DC = 1024
ND = DICT // DC
TBLKS = TOKENS // TB


def _sae_kernel(x_ref, w_enc_ref, w_dec_ref, b_dec_ref,
                sae_out_ref, acts_topk_ref, sae_err_ref, part_ref,
                acts_s, bits16_s, xn_s, xe_s, mean_s, std_s, tau_s, xrec_s,
                l1r_s, l0r_s):
    s = pl.program_id(1)

    @pl.when(s == 0)
    def _normalize():
        xb = x_ref[...]
        mean = jnp.mean(xb, axis=1, keepdims=True)
        xc = xb - mean
        var = jnp.sum(xc * xc, axis=1, keepdims=True) * (1.0 / (ACT - 1))
        std = jnp.sqrt(var)
        xn = xc / (std + 1e-5)
        mean_s[...] = mean
        std_s[...] = std
        xn_s[...] = xn
        xe_s[...] = xn - b_dec_ref[...]

    @pl.when(s < ND)
    def _encode():
        z = jnp.dot(xe_s[...], w_enc_ref[...],
                    preferred_element_type=jnp.float32)
        a = jnp.maximum(z, 0.0)
        acts_s[:, pl.ds(s * DC, DC)] = a
        hi = jnp.right_shift(lax.bitcast_convert_type(a, jnp.int32), 16)
        bits16_s[:, pl.ds(s * DC, DC)] = hi.astype(jnp.int16)

    @pl.when(s == ND)
    def _threshold():
        def body16(i, lo):
            bitpos = 14 - i
            t = lo | jnp.left_shift(jnp.int32(1), bitpos)
            t16 = t.astype(jnp.int16)
            cnt = jnp.sum((bits16_s[...] >= t16).astype(jnp.int32), axis=1,
                          keepdims=True)
            return jnp.where(cnt >= TOPK, t, lo)

        lo16 = jnp.zeros((TB, 1), jnp.int32)
        lo16 = lax.fori_loop(0, 15, body16, lo16)
        lo_init = jnp.left_shift(lo16, 16)

        def body(i, lo):
            bitpos = 15 - i
            t = lo | jnp.left_shift(jnp.int32(1), bitpos)
            bits = lax.bitcast_convert_type(acts_s[...], jnp.int32)
            cnt = jnp.sum((bits >= t).astype(jnp.int32), axis=1, keepdims=True)
            return jnp.where(cnt >= TOPK, t, lo)

        lo = lax.fori_loop(0, 16, body, lo_init)
        tau_s[...] = lax.bitcast_convert_type(lo, jnp.float32)

    @pl.when(s >= ND)
    def _mask_decode():
        c = s - ND
        acts = acts_s[:, pl.ds(c * DC, DC)]
        atk = jnp.where(acts >= tau_s[...], acts, 0.0)
        acts_topk_ref[...] = atk
        part = jnp.dot(atk, w_dec_ref[...], preferred_element_type=jnp.float32)
        l1c = jnp.sum(atk, axis=1, keepdims=True)
        l0c = jnp.sum((atk > 0).astype(jnp.float32), axis=1, keepdims=True)

        @pl.when(s == ND)
        def _init():
            xrec_s[...] = part
            l1r_s[...] = l1c
            l0r_s[...] = l0c

        @pl.when(s > ND)
        def _acc():
            xrec_s[...] = xrec_s[...] + part
            l1r_s[...] = l1r_s[...] + l1c
            l0r_s[...] = l0r_s[...] + l0c

    @pl.when(s == 2 * ND - 1)
    def _finalize():
        xrec = xrec_s[...] + b_dec_ref[...]
        std = std_s[...]
        mean = mean_s[...]
        xn = xn_s[...]
        sae_out = xrec * std + mean
        sae_out_ref[...] = sae_out
        sae_err_ref[...] = (xn * std + mean) - sae_out
        diff = xrec - xn
        l2p = jnp.sum(diff * diff)
        l1p = jnp.sum(l1r_s[...])
        l0p = jnp.sum(l0r_s[...])
        lane = lax.broadcasted_iota(jnp.int32, (1, 1, 128), 2)
        part_ref[...] = jnp.where(
            lane == 0, l2p, jnp.where(lane == 1, l1p,
                                      jnp.where(lane == 2, l0p, 0.0)))


@functools.partial(jax.jit)
def _run(xs, W_enc, W_dec, b_dec2):
    grid = (TBLKS, 2 * ND)
    out_shapes = (
        jax.ShapeDtypeStruct((TOKENS, ACT), jnp.float32),
        jax.ShapeDtypeStruct((TOKENS, DICT), jnp.float32),
        jax.ShapeDtypeStruct((TOKENS, ACT), jnp.float32),
        jax.ShapeDtypeStruct((TBLKS, 1, 128), jnp.float32),
    )
    in_specs = [
        pl.BlockSpec((TB, ACT), lambda t, s: (t, 0)),
        pl.BlockSpec((ACT, DC), lambda t, s: (0, jnp.minimum(s, ND - 1))),
        pl.BlockSpec((DC, ACT), lambda t, s: (jnp.maximum(s - ND, 0), 0)),
        pl.BlockSpec((1, ACT), lambda t, s: (0, 0)),
    ]
    out_specs = (
        pl.BlockSpec((TB, ACT), lambda t, s: (t, 0)),
        pl.BlockSpec((TB, DC), lambda t, s: (t, jnp.maximum(s - ND, 0))),
        pl.BlockSpec((TB, ACT), lambda t, s: (t, 0)),
        pl.BlockSpec((1, 1, 128), lambda t, s: (t, 0, 0)),
    )
    scratch = [
        pltpu.VMEM((TB, DICT), jnp.float32),
        pltpu.VMEM((TB, DICT), jnp.int16),
        pltpu.VMEM((TB, ACT), jnp.float32),
        pltpu.VMEM((TB, ACT), jnp.float32),
        pltpu.VMEM((TB, 1), jnp.float32),
        pltpu.VMEM((TB, 1), jnp.float32),
        pltpu.VMEM((TB, 1), jnp.float32),
        pltpu.VMEM((TB, ACT), jnp.float32),
        pltpu.VMEM((TB, 1), jnp.float32),
        pltpu.VMEM((TB, 1), jnp.float32),
    ]
    return pl.pallas_call(
        _sae_kernel,
        grid=grid,
        in_specs=in_specs,
        out_specs=out_specs,
        out_shape=out_shapes,
        scratch_shapes=scratch,
        compiler_params=pltpu.CompilerParams(
            dimension_semantics=("parallel", "arbitrary"),
        ),
    )(xs, W_enc, W_dec, b_dec2)


def kernel(x, W_enc, W_dec, b_dec):
    xs = x[0]
    b_dec2 = b_dec.reshape(1, ACT)
    sae_out, acts_topk, sae_error, parts = _run(xs, W_enc, W_dec, b_dec2)
    l2_sum = jnp.sum(parts[:, 0, 0])
    l1_sum = jnp.sum(parts[:, 0, 1])
    l0_sum = jnp.sum(parts[:, 0, 2])
    l2_loss = l2_sum / (TOKENS * ACT)
    l1_norm = l1_sum / TOKENS
    l0_norm = l0_sum / TOKENS
    l1_loss = L1_COEFF * l1_norm
    loss = l2_loss
    return sae_out, acts_topk, loss, l1_loss, l2_loss, l0_norm, l1_norm, sae_error


# speedup vs baseline: 1.2999x; 1.2999x over previous
import functools

import jax
import jax.numpy as jnp
from jax import lax
from jax.experimental import pallas as pl
from jax.experimental.pallas import tpu as pltpu

ACT = 1024
DICT = 16384
TOKENS = 4096
TOPK = 64
L1_COEFF = 0.0008

TB = 512
DC = 512
ND = DICT // DC
TBLKS = TOKENS // TB


def _norm_stats(xb):
    mean = jnp.mean(xb, axis=1, keepdims=True)
    xc = xb - mean
    var = jnp.sum(xc * xc, axis=1, keepdims=True) * (1.0 / (ACT - 1))
    std = jnp.sqrt(var)
    xn = xc / (std + 1e-5)
    return mean, std, xn


def _sae_kernel(x_ref, w_enc_ref, w_dec_ref, b_dec_ref,
                sae_out_ref, acts_topk_ref, sae_err_ref, part_ref,
                acts_s, misc_s):
    s = pl.program_id(1)

    @pl.when(s == 0)
    def _normalize():
        _, _, xn = _norm_stats(x_ref[...])
        sae_err_ref[...] = xn - b_dec_ref[...]

    @pl.when(s < ND)
    def _encode():
        z = jnp.dot(sae_err_ref[...], w_enc_ref[...],
                    preferred_element_type=jnp.float32)
        acts_s[:, pl.ds(s * DC, DC)] = jnp.maximum(z, 0.0)

    @pl.when(s == ND)
    def _threshold():
        def body(i, lo):
            bitpos = 30 - i
            t = lo | jnp.left_shift(jnp.int32(1), bitpos)
            bits = lax.bitcast_convert_type(acts_s[...], jnp.int32)
            cnt = jnp.sum((bits >= t).astype(jnp.int32), axis=1, keepdims=True)
            return jnp.where(cnt >= TOPK, t, lo)

        lo = jnp.zeros((TB, 1), jnp.int32)
        lo = lax.fori_loop(0, 31, body, lo)
        misc_s[:, 0:1] = lax.bitcast_convert_type(lo, jnp.float32)

    @pl.when(s >= ND)
    def _mask_decode():
        c = s - ND
        acts = acts_s[:, pl.ds(c * DC, DC)]
        atk = jnp.where(acts >= misc_s[:, 0:1], acts, 0.0)
        acts_topk_ref[...] = atk
        part = jnp.dot(atk, w_dec_ref[...], preferred_element_type=jnp.float32)
        l1c = jnp.sum(atk, axis=1, keepdims=True)
        l0c = jnp.sum((atk > 0).astype(jnp.float32), axis=1, keepdims=True)

        @pl.when(s == ND)
        def _init():
            sae_out_ref[...] = part
            misc_s[:, 1:2] = l1c
            misc_s[:, 2:3] = l0c

        @pl.when(s > ND)
        def _acc():
            sae_out_ref[...] = sae_out_ref[...] + part
            misc_s[:, 1:2] = misc_s[:, 1:2] + l1c
            misc_s[:, 2:3] = misc_s[:, 2:3] + l0c

    @pl.when(s == 2 * ND - 1)
    def _finalize():
        mean, std, xn = _norm_stats(x_ref[...])
        xrec = sae_out_ref[...] + b_dec_ref[...]
        sae_out = xrec * std + mean
        sae_out_ref[...] = sae_out
        sae_err_ref[...] = (xn * std + mean) - sae_out
        diff = xrec - xn
        l2p = jnp.sum(diff * diff)
        l1p = jnp.sum(misc_s[:, 1:2])
        l0p = jnp.sum(misc_s[:, 2:3])
        lane = lax.broadcasted_iota(jnp.int32, (1, 1, 128), 2)
        part_ref[...] = jnp.where(
            lane == 0, l2p, jnp.where(lane == 1, l1p,
                                      jnp.where(lane == 2, l0p, 0.0)))


@functools.partial(jax.jit)
def _run(xs, W_enc, W_dec, b_dec2):
    grid = (TBLKS, 2 * ND)
    out_shapes = (
        jax.ShapeDtypeStruct((TOKENS, ACT), jnp.float32),
        jax.ShapeDtypeStruct((TOKENS, DICT), jnp.float32),
        jax.ShapeDtypeStruct((TOKENS, ACT), jnp.float32),
        jax.ShapeDtypeStruct((TBLKS, 1, 128), jnp.float32),
    )
    in_specs = [
        pl.BlockSpec((TB, ACT), lambda t, s: (t, 0)),
        pl.BlockSpec((ACT, DC), lambda t, s: (0, jnp.minimum(s, ND - 1))),
        pl.BlockSpec((DC, ACT), lambda t, s: (jnp.maximum(s - ND, 0), 0)),
        pl.BlockSpec((1, ACT), lambda t, s: (0, 0)),
    ]
    out_specs = (
        pl.BlockSpec((TB, ACT), lambda t, s: (t, 0)),
        pl.BlockSpec((TB, DC), lambda t, s: (t, jnp.maximum(s - ND, 0))),
        pl.BlockSpec((TB, ACT), lambda t, s: (t, 0)),
        pl.BlockSpec((1, 1, 128), lambda t, s: (t, 0, 0)),
    )
    scratch = [
        pltpu.VMEM((TB, DICT), jnp.float32),
        pltpu.VMEM((TB, 128), jnp.float32),
    ]
    return pl.pallas_call(
        _sae_kernel,
        grid=grid,
        in_specs=in_specs,
        out_specs=out_specs,
        out_shape=out_shapes,
        scratch_shapes=scratch,
        compiler_params=pltpu.CompilerParams(
            dimension_semantics=("arbitrary", "arbitrary"),
        ),
    )(xs, W_enc, W_dec, b_dec2)


def kernel(x, W_enc, W_dec, b_dec):
    xs = x[0]
    b_dec2 = b_dec.reshape(1, ACT)
    sae_out, acts_topk, sae_error, parts = _run(xs, W_enc, W_dec, b_dec2)
    l2_sum = jnp.sum(parts[:, 0, 0])
    l1_sum = jnp.sum(parts[:, 0, 1])
    l0_sum = jnp.sum(parts[:, 0, 2])
    l2_loss = l2_sum / (TOKENS * ACT)
    l1_norm = l1_sum / TOKENS
    l0_norm = l0_sum / TOKENS
    l1_loss = L1_COEFF * l1_norm
    loss = l2_loss
    return sae_out, acts_topk, loss, l1_loss, l2_loss, l0_norm, l1_norm, sae_error
